# baseline (speedup 1.0000x reference)
import functools

import jax
import jax.numpy as jnp
from jax import lax
from jax.experimental import pallas as pl

_LS = 1
_LE = 1
_EPS = 1e-10
_C = 768


def _fused_tc_kernel(L, C, HB, a1_ref, a2_ref, s1_ref, s2_ref, mt_ref,
                     ao_ref, so_ref):
    P = C - L
    b = pl.program_id(0)
    h = pl.program_id(1)

    r = lax.broadcasted_iota(jnp.int32, (L, L), 0)
    q = lax.broadcasted_iota(jnp.int32, (L, L), 1)
    keep1 = (r < L - _LE) & (q < L - _LE)
    keep2 = (r >= _LS) & (q >= _LS)
    for hh in range(HB):
        a1m = jnp.where(keep1, a1_ref[0, hh], 0.0)
        a2m = jnp.where(keep2, a2_ref[0, hh], 0.0)
        acc = jnp.pad(a1m, ((0, P), (0, P))) + jnp.pad(a2m, ((P, 0), (P, 0)))
        s = acc.sum(axis=-1, keepdims=True)
        ao_ref[0, hh] = acc * (1.0 / (s + _EPS))

    @pl.when(h == 0)
    def _():
        s1 = s1_ref[0]
        s2 = s2_ref[0]
        rr = lax.broadcasted_iota(jnp.int32, (L, 1), 0)
        k1 = rr < L - _LE
        k2 = rr >= _LS
        mt = mt_ref[:]
        col = lax.broadcasted_iota(jnp.int32, mt.shape, 1)
        mc1 = jnp.sum(jnp.where(col == 2 * b, mt, 0.0), axis=1, keepdims=True)
        mc2 = jnp.sum(jnp.where(col == 2 * b + 1, mt, 0.0), axis=1, keepdims=True)
        m1 = jnp.where(k1, mc1, 0.0)
        m2 = jnp.where(k2, mc2, 0.0)
        s1m = jnp.where(k1, s1, 0.0)
        s2m = jnp.where(k2, s2, 0.0)
        acc = jnp.pad(s1m, ((0, P), (0, 0))) + jnp.pad(s2m, ((P, 0), (0, 0)))
        mv = jnp.pad(m1, ((0, P), (0, 0))) + jnp.pad(m2, ((P, 0), (0, 0))) + _EPS
        so_ref[0] = acc * (1.0 / mv)


def kernel(sequence_output, attention, chunk_attention_mask, num_seg, seq_len, orig_c):
    NC, L, D = sequence_output.shape
    H = attention.shape[1]
    Bb = NC // 2
    c = _C
    if c <= L:
        return (sequence_output, attention)

    HB = 4
    mt = chunk_attention_mask.astype(jnp.float32).T
    new_attention, new_output = pl.pallas_call(
        functools.partial(_fused_tc_kernel, L, c, HB),
        grid=(Bb, H // HB),
        in_specs=[
            pl.BlockSpec((1, HB, L, L), lambda b, h: (2 * b, h, 0, 0)),
            pl.BlockSpec((1, HB, L, L), lambda b, h: (2 * b + 1, h, 0, 0)),
            pl.BlockSpec((1, L, D), lambda b, h: (2 * b, 0, 0)),
            pl.BlockSpec((1, L, D), lambda b, h: (2 * b + 1, 0, 0)),
            pl.BlockSpec((L, NC), lambda b, h: (0, 0)),
        ],
        out_specs=[
            pl.BlockSpec((1, HB, c, c), lambda b, h: (b, h, 0, 0)),
            pl.BlockSpec((1, c, D), lambda b, h: (b, 0, 0)),
        ],
        out_shape=[
            jax.ShapeDtypeStruct((Bb, H, c, c), jnp.float32),
            jax.ShapeDtypeStruct((Bb, c, D), jnp.float32),
        ],
    )(attention, attention, sequence_output, sequence_output, mt)

    return (new_output, new_attention)


# speedup vs baseline: 1.0002x; 1.0002x over previous
import functools

import jax
import jax.numpy as jnp
from jax import lax
from jax.experimental import pallas as pl
from jax.experimental.pallas import tpu as pltpu

_LS = 1
_LE = 1
_EPS = 1e-10
_C = 768


def _fused_tc_kernel(L, C, HB, a1_ref, a2_ref, s1_ref, s2_ref, mt_ref,
                     ao_ref, so_ref):
    P = C - L
    b = pl.program_id(0)
    h = pl.program_id(1)

    r = lax.broadcasted_iota(jnp.int32, (L, L), 0)
    q = lax.broadcasted_iota(jnp.int32, (L, L), 1)
    keep1 = (r < L - _LE) & (q < L - _LE)
    keep2 = (r >= _LS) & (q >= _LS)
    for hh in range(HB):
        a1m = jnp.where(keep1, a1_ref[0, hh], 0.0)
        a2m = jnp.where(keep2, a2_ref[0, hh], 0.0)
        acc = jnp.pad(a1m, ((0, P), (0, P))) + jnp.pad(a2m, ((P, 0), (P, 0)))
        s = acc.sum(axis=-1, keepdims=True)
        ao_ref[0, hh] = acc * (1.0 / (s + _EPS))

    @pl.when(h == 0)
    def _():
        s1 = s1_ref[0]
        s2 = s2_ref[0]
        rr = lax.broadcasted_iota(jnp.int32, (L, 1), 0)
        k1 = rr < L - _LE
        k2 = rr >= _LS
        mt = mt_ref[:]
        col = lax.broadcasted_iota(jnp.int32, mt.shape, 1)
        mc1 = jnp.sum(jnp.where(col == 2 * b, mt, 0.0), axis=1, keepdims=True)
        mc2 = jnp.sum(jnp.where(col == 2 * b + 1, mt, 0.0), axis=1, keepdims=True)
        m1 = jnp.where(k1, mc1, 0.0)
        m2 = jnp.where(k2, mc2, 0.0)
        s1m = jnp.where(k1, s1, 0.0)
        s2m = jnp.where(k2, s2, 0.0)
        acc = jnp.pad(s1m, ((0, P), (0, 0))) + jnp.pad(s2m, ((P, 0), (0, 0)))
        mv = jnp.pad(m1, ((0, P), (0, 0))) + jnp.pad(m2, ((P, 0), (0, 0))) + _EPS
        so_ref[0] = acc * (1.0 / mv)


def kernel(sequence_output, attention, chunk_attention_mask, num_seg, seq_len, orig_c):
    NC, L, D = sequence_output.shape
    H = attention.shape[1]
    Bb = NC // 2
    c = _C
    if c <= L:
        return (sequence_output, attention)

    HB = 4
    mt = chunk_attention_mask.astype(jnp.float32).T
    new_attention, new_output = pl.pallas_call(
        functools.partial(_fused_tc_kernel, L, c, HB),
        grid=(Bb, H // HB),
        in_specs=[
            pl.BlockSpec((1, HB, L, L), lambda b, h: (2 * b, h, 0, 0)),
            pl.BlockSpec((1, HB, L, L), lambda b, h: (2 * b + 1, h, 0, 0)),
            pl.BlockSpec((1, L, D), lambda b, h: (2 * b, 0, 0)),
            pl.BlockSpec((1, L, D), lambda b, h: (2 * b + 1, 0, 0)),
            pl.BlockSpec((L, NC), lambda b, h: (0, 0)),
        ],
        out_specs=[
            pl.BlockSpec((1, HB, c, c), lambda b, h: (b, h, 0, 0)),
            pl.BlockSpec((1, c, D), lambda b, h: (b, 0, 0)),
        ],
        out_shape=[
            jax.ShapeDtypeStruct((Bb, H, c, c), jnp.float32),
            jax.ShapeDtypeStruct((Bb, c, D), jnp.float32),
        ],
        compiler_params=pltpu.CompilerParams(
            dimension_semantics=("parallel", "arbitrary")),
    )(attention, attention, sequence_output, sequence_output, mt)

    return (new_output, new_attention)
